# initial kernel scaffold (unmeasured)
import jax
import jax.numpy as jnp
from jax import lax
from jax.experimental import pallas as pl
from jax.experimental.pallas import tpu as pltpu

B, SQ, H, D = 4, 256, 16, 64
SCALE = D ** -0.5


def _comm_body(k_ref, v_ref, krecv_ref, vrecv_ref,
               ksend, vsend, send_sems, recv_sems):
    my_x = lax.axis_index("x")
    my_y = lax.axis_index("y")
    partner = (1 - my_x, my_y)

    barrier_sem = pltpu.get_barrier_semaphore()
    pl.semaphore_signal(barrier_sem, inc=1, device_id=partner,
                        device_id_type=pl.DeviceIdType.MESH)
    pl.semaphore_wait(barrier_sem, 1)

    ksend[...] = k_ref[...].astype(jnp.bfloat16)
    vsend[...] = v_ref[...].astype(jnp.bfloat16)

    rk = pltpu.make_async_remote_copy(
        src_ref=ksend, dst_ref=krecv_ref,
        send_sem=send_sems.at[0], recv_sem=recv_sems.at[0],
        device_id=partner, device_id_type=pl.DeviceIdType.MESH,
    )
    rv = pltpu.make_async_remote_copy(
        src_ref=vsend, dst_ref=vrecv_ref,
        send_sem=send_sems.at[1], recv_sem=recv_sems.at[1],
        device_id=partner, device_id_type=pl.DeviceIdType.MESH,
    )
    rk.start()
    rv.start()
    rk.wait()
    rv.wait()


def _attn_body(q_ref, k_ref, v_ref, krecv_ref, vrecv_ref, o_ref):
    q = q_ref[0, :, 0, :].astype(jnp.bfloat16)
    k0 = k_ref[0, :, 0, :].astype(jnp.bfloat16)
    k1 = krecv_ref[0, :, 0, :]
    v0 = v_ref[0, :, 0, :].astype(jnp.bfloat16)
    v1 = vrecv_ref[0, :, 0, :]

    k = jnp.concatenate([k0, k1], axis=0)
    v = jnp.concatenate([v0, v1], axis=0)

    s = lax.dot_general(q, k, (((1,), (1,)), ((), ())),
                        preferred_element_type=jnp.float32) * SCALE
    m = jnp.max(s, axis=1, keepdims=True)
    e = jnp.exp(s - m)
    p = e / jnp.sum(e, axis=1, keepdims=True)
    o = lax.dot_general(p.astype(jnp.bfloat16), v, (((1,), (0,)), ((), ())),
                        preferred_element_type=jnp.float32)
    o_ref[0, :, 0, :] = o


def kernel(Q, K, V):
    krecv, vrecv = pl.pallas_call(
        _comm_body,
        out_shape=[
            jax.ShapeDtypeStruct((B, SQ, H, D), jnp.bfloat16),
            jax.ShapeDtypeStruct((B, SQ, H, D), jnp.bfloat16),
        ],
        in_specs=[
            pl.BlockSpec(memory_space=pltpu.VMEM),
            pl.BlockSpec(memory_space=pltpu.VMEM),
        ],
        out_specs=[
            pl.BlockSpec(memory_space=pltpu.VMEM),
            pl.BlockSpec(memory_space=pltpu.VMEM),
        ],
        scratch_shapes=[
            pltpu.VMEM((B, SQ, H, D), jnp.bfloat16),
            pltpu.VMEM((B, SQ, H, D), jnp.bfloat16),
            pltpu.SemaphoreType.DMA((2,)),
            pltpu.SemaphoreType.DMA((2,)),
        ],
        compiler_params=pltpu.CompilerParams(collective_id=0),
    )(K, V)

    blk = pl.BlockSpec((1, SQ, 1, D), lambda b, h: (b, 0, h, 0))
    return pl.pallas_call(
        _attn_body,
        grid=(B, H),
        out_shape=jax.ShapeDtypeStruct((B, SQ, H, D), jnp.float32),
        in_specs=[blk, blk, blk, blk, blk],
        out_specs=blk,
    )(Q, K, V, krecv, vrecv)


# baseline (device time: 185969 ns/iter reference)
import jax
import jax.numpy as jnp
from jax import lax
from jax.experimental import pallas as pl
from jax.experimental.pallas import tpu as pltpu

B, SQ, H, D = 4, 256, 16, 64
SCALE = D ** -0.5


def _comm_body(k_ref, v_ref, krecv_ref, vrecv_ref,
               ksend, vsend, send_sems, recv_sems):
    my_x = lax.axis_index("x")
    my_y = lax.axis_index("y")
    partner = (1 - my_x, my_y)

    barrier_sem = pltpu.get_barrier_semaphore()
    pl.semaphore_signal(barrier_sem, inc=1, device_id=partner,
                        device_id_type=pl.DeviceIdType.MESH)
    pl.semaphore_wait(barrier_sem, 1)

    ksend[...] = k_ref[...].astype(jnp.bfloat16)
    vsend[...] = v_ref[...].astype(jnp.bfloat16)

    rk = pltpu.make_async_remote_copy(
        src_ref=ksend, dst_ref=krecv_ref,
        send_sem=send_sems.at[0], recv_sem=recv_sems.at[0],
        device_id=partner, device_id_type=pl.DeviceIdType.MESH,
    )
    rv = pltpu.make_async_remote_copy(
        src_ref=vsend, dst_ref=vrecv_ref,
        send_sem=send_sems.at[1], recv_sem=recv_sems.at[1],
        device_id=partner, device_id_type=pl.DeviceIdType.MESH,
    )
    rk.start()
    rv.start()
    rk.wait()
    rv.wait()


def _attn_body(q_ref, k_ref, v_ref, krecv_ref, vrecv_ref, o_ref):
    for h in range(H):
        q = q_ref[0, :, h, :].astype(jnp.bfloat16)
        k0 = k_ref[0, :, h, :].astype(jnp.bfloat16)
        k1 = krecv_ref[0, :, h, :]
        v0 = v_ref[0, :, h, :].astype(jnp.bfloat16)
        v1 = vrecv_ref[0, :, h, :]

        k = jnp.concatenate([k0, k1], axis=0)
        v = jnp.concatenate([v0, v1], axis=0)

        s = lax.dot_general(q, k, (((1,), (1,)), ((), ())),
                            preferred_element_type=jnp.float32) * SCALE
        m = jnp.max(s, axis=1, keepdims=True)
        e = jnp.exp(s - m)
        p = e / jnp.sum(e, axis=1, keepdims=True)
        o = lax.dot_general(p.astype(jnp.bfloat16), v,
                            (((1,), (0,)), ((), ())),
                            preferred_element_type=jnp.float32)
        o_ref[0, :, h, :] = o


def kernel(Q, K, V):
    krecv, vrecv = pl.pallas_call(
        _comm_body,
        out_shape=[
            jax.ShapeDtypeStruct((B, SQ, H, D), jnp.bfloat16),
            jax.ShapeDtypeStruct((B, SQ, H, D), jnp.bfloat16),
        ],
        in_specs=[
            pl.BlockSpec(memory_space=pltpu.VMEM),
            pl.BlockSpec(memory_space=pltpu.VMEM),
        ],
        out_specs=[
            pl.BlockSpec(memory_space=pltpu.VMEM),
            pl.BlockSpec(memory_space=pltpu.VMEM),
        ],
        scratch_shapes=[
            pltpu.VMEM((B, SQ, H, D), jnp.bfloat16),
            pltpu.VMEM((B, SQ, H, D), jnp.bfloat16),
            pltpu.SemaphoreType.DMA((2,)),
            pltpu.SemaphoreType.DMA((2,)),
        ],
        compiler_params=pltpu.CompilerParams(collective_id=0),
    )(K, V)

    blk = pl.BlockSpec((1, SQ, H, D), lambda b: (b, 0, 0, 0))
    return pl.pallas_call(
        _attn_body,
        grid=(B,),
        out_shape=jax.ShapeDtypeStruct((B, SQ, H, D), jnp.float32),
        in_specs=[blk, blk, blk, blk, blk],
        out_specs=blk,
    )(Q, K, V, krecv, vrecv)


# device time: 172467 ns/iter; 1.0783x vs baseline; 1.0783x over previous
import jax
import jax.numpy as jnp
from jax import lax
from jax.experimental import pallas as pl
from jax.experimental.pallas import tpu as pltpu

B, SQ, H, D = 4, 256, 16, 64
SCALE = D ** -0.5


def _comm_body(k_ref, v_ref, krecv_ref, vrecv_ref,
               ksend, vsend, send_sems, recv_sems):
    my_x = lax.axis_index("x")
    my_y = lax.axis_index("y")
    partner = (1 - my_x, my_y)

    barrier_sem = pltpu.get_barrier_semaphore()
    pl.semaphore_signal(barrier_sem, inc=1, device_id=partner,
                        device_id_type=pl.DeviceIdType.MESH)
    pl.semaphore_wait(barrier_sem, 1)

    ksend[...] = k_ref[...].astype(jnp.bfloat16)
    vsend[...] = v_ref[...].astype(jnp.bfloat16)

    rk = pltpu.make_async_remote_copy(
        src_ref=ksend, dst_ref=krecv_ref,
        send_sem=send_sems.at[0], recv_sem=recv_sems.at[0],
        device_id=partner, device_id_type=pl.DeviceIdType.MESH,
    )
    rv = pltpu.make_async_remote_copy(
        src_ref=vsend, dst_ref=vrecv_ref,
        send_sem=send_sems.at[1], recv_sem=recv_sems.at[1],
        device_id=partner, device_id_type=pl.DeviceIdType.MESH,
    )
    rk.start()
    rv.start()
    rk.wait()
    rv.wait()


def _attn_body(q_ref, k_ref, v_ref, krecv_ref, vrecv_ref, o_ref):
    q = q_ref[0, 0].astype(jnp.bfloat16)
    k0 = k_ref[0, 0].astype(jnp.bfloat16)
    k1 = krecv_ref[0, 0]
    v0 = v_ref[0, 0].astype(jnp.bfloat16)
    v1 = vrecv_ref[0, 0]

    k = jnp.concatenate([k0, k1], axis=0)
    v = jnp.concatenate([v0, v1], axis=0)

    s = lax.dot_general(q, k, (((1,), (1,)), ((), ())),
                        preferred_element_type=jnp.float32) * SCALE
    m = jnp.max(s, axis=1, keepdims=True)
    e = jnp.exp(s - m)
    p = e / jnp.sum(e, axis=1, keepdims=True)
    o = lax.dot_general(p.astype(jnp.bfloat16), v, (((1,), (0,)), ((), ())),
                        preferred_element_type=jnp.float32)
    o_ref[0, 0] = o


def kernel(Q, K, V):
    Qt = jnp.transpose(Q, (0, 2, 1, 3))
    Kt = jnp.transpose(K, (0, 2, 1, 3))
    Vt = jnp.transpose(V, (0, 2, 1, 3))

    krecv, vrecv = pl.pallas_call(
        _comm_body,
        out_shape=[
            jax.ShapeDtypeStruct((B, H, SQ, D), jnp.bfloat16),
            jax.ShapeDtypeStruct((B, H, SQ, D), jnp.bfloat16),
        ],
        in_specs=[
            pl.BlockSpec(memory_space=pltpu.VMEM),
            pl.BlockSpec(memory_space=pltpu.VMEM),
        ],
        out_specs=[
            pl.BlockSpec(memory_space=pltpu.VMEM),
            pl.BlockSpec(memory_space=pltpu.VMEM),
        ],
        scratch_shapes=[
            pltpu.VMEM((B, H, SQ, D), jnp.bfloat16),
            pltpu.VMEM((B, H, SQ, D), jnp.bfloat16),
            pltpu.SemaphoreType.DMA((2,)),
            pltpu.SemaphoreType.DMA((2,)),
        ],
        compiler_params=pltpu.CompilerParams(collective_id=0),
    )(Kt, Vt)

    blk = pl.BlockSpec((1, 1, SQ, D), lambda b, h: (b, h, 0, 0))
    out = pl.pallas_call(
        _attn_body,
        grid=(B, H),
        out_shape=jax.ShapeDtypeStruct((B, H, SQ, D), jnp.float32),
        in_specs=[blk, blk, blk, blk, blk],
        out_specs=blk,
    )(Qt, Kt, Vt, krecv, vrecv)
    return jnp.transpose(out, (0, 2, 1, 3))


# device time: 103366 ns/iter; 1.7991x vs baseline; 1.6685x over previous
import jax
import jax.numpy as jnp
from jax import lax
from jax.experimental import pallas as pl
from jax.experimental.pallas import tpu as pltpu

B, SQ, H, D = 4, 256, 16, 64
HH = H // 2
SCALE = D ** -0.5
NC = 2 * B


def _fused_body(q_ref, k_ref, v_ref, o_ref,
                ksend, vsend, krecv, vrecv,
                xsend_sems, xrecv_sems, ysend_sems, yrecv_sems):
    my_x = lax.axis_index("x")
    my_y = lax.axis_index("y")
    partner = (1 - my_x, my_y)
    peer = (my_x, 1 - my_y)
    hh = HH * my_y

    barrier_sem = pltpu.get_barrier_semaphore()
    for nbr in (partner, peer):
        pl.semaphore_signal(barrier_sem, inc=1, device_id=nbr,
                            device_id_type=pl.DeviceIdType.MESH)
    pl.semaphore_wait(barrier_sem, 2)

    send_bufs = (ksend, vsend)
    recv_bufs = (krecv, vrecv)
    src_refs = (k_ref, v_ref)

    x_rdmas = []
    for b in range(B):
        for t in range(2):
            c = 2 * b + t
            send_bufs[t][b] = src_refs[t][b].astype(jnp.bfloat16)
            rdma = pltpu.make_async_remote_copy(
                src_ref=send_bufs[t].at[b, pl.ds(hh, HH)],
                dst_ref=recv_bufs[t].at[b, pl.ds(hh, HH)],
                send_sem=xsend_sems.at[c], recv_sem=xrecv_sems.at[c],
                device_id=partner, device_id_type=pl.DeviceIdType.MESH,
            )
            rdma.start()
            x_rdmas.append(rdma)

    y_rdmas = []
    for b in range(B):
        for t in range(2):
            c = 2 * b + t
            x_rdmas[c].wait_recv()
            fwd = pltpu.make_async_remote_copy(
                src_ref=recv_bufs[t].at[b, pl.ds(hh, HH)],
                dst_ref=recv_bufs[t].at[b, pl.ds(hh, HH)],
                send_sem=ysend_sems.at[c], recv_sem=yrecv_sems.at[c],
                device_id=peer, device_id_type=pl.DeviceIdType.MESH,
            )
            fwd.start()
            y_rdmas.append(fwd)
        for t in range(2):
            y_rdmas[2 * b + t].wait_recv()

        for h in range(H):
            q = (q_ref[b, h] * SCALE).astype(jnp.bfloat16)
            k0, v0 = ksend[b, h], vsend[b, h]
            k1, v1 = krecv[b, h], vrecv[b, h]
            s0 = lax.dot_general(q, k0, (((1,), (1,)), ((), ())),
                                 preferred_element_type=jnp.float32)
            s1 = lax.dot_general(q, k1, (((1,), (1,)), ((), ())),
                                 preferred_element_type=jnp.float32)
            e0 = jnp.exp(s0)
            e1 = jnp.exp(s1)
            l = jnp.sum(e0, 1, keepdims=True) + jnp.sum(e1, 1, keepdims=True)
            o = (lax.dot_general(e0.astype(jnp.bfloat16), v0,
                                 (((1,), (0,)), ((), ())),
                                 preferred_element_type=jnp.float32)
                 + lax.dot_general(e1.astype(jnp.bfloat16), v1,
                                   (((1,), (0,)), ((), ())),
                                   preferred_element_type=jnp.float32))
            o_ref[b, h] = o / l

    for r in x_rdmas:
        r.wait_send()
    for r in y_rdmas:
        r.wait_send()


def kernel(Q, K, V):
    Qt = jnp.transpose(Q, (0, 2, 1, 3))
    Kt = jnp.transpose(K, (0, 2, 1, 3))
    Vt = jnp.transpose(V, (0, 2, 1, 3))

    vm = pl.BlockSpec(memory_space=pltpu.VMEM)
    out = pl.pallas_call(
        _fused_body,
        out_shape=jax.ShapeDtypeStruct((B, H, SQ, D), jnp.float32),
        in_specs=[vm, vm, vm],
        out_specs=vm,
        scratch_shapes=[
            pltpu.VMEM((B, H, SQ, D), jnp.bfloat16),
            pltpu.VMEM((B, H, SQ, D), jnp.bfloat16),
            pltpu.VMEM((B, H, SQ, D), jnp.bfloat16),
            pltpu.VMEM((B, H, SQ, D), jnp.bfloat16),
            pltpu.SemaphoreType.DMA((NC,)),
            pltpu.SemaphoreType.DMA((NC,)),
            pltpu.SemaphoreType.DMA((NC,)),
            pltpu.SemaphoreType.DMA((NC,)),
        ],
        compiler_params=pltpu.CompilerParams(collective_id=0),
    )(Qt, Kt, Vt)
    return jnp.transpose(out, (0, 2, 1, 3))


# device time: 67456 ns/iter; 2.7569x vs baseline; 1.5323x over previous
import jax
import jax.numpy as jnp
from jax import lax
from jax.experimental import pallas as pl
from jax.experimental.pallas import tpu as pltpu

B, SQ, H, D = 4, 256, 16, 64
HD = H * D
HL = HD // 2
SCALE = D ** -0.5
NC = 2 * B


def _fused_body(q_ref, k_ref, v_ref, o_ref,
                ksend, vsend, krecv, vrecv,
                xsend_sems, xrecv_sems, ysend_sems, yrecv_sems):
    my_x = lax.axis_index("x")
    my_y = lax.axis_index("y")
    partner = (1 - my_x, my_y)
    peer = (my_x, 1 - my_y)
    hl = HL * my_y

    barrier_sem = pltpu.get_barrier_semaphore()
    for nbr in (partner, peer):
        pl.semaphore_signal(barrier_sem, inc=1, device_id=nbr,
                            device_id_type=pl.DeviceIdType.MESH)
    pl.semaphore_wait(barrier_sem, 2)

    send_bufs = (ksend, vsend)
    recv_bufs = (krecv, vrecv)
    src_refs = (k_ref, v_ref)

    x_rdmas = []
    for b in range(B):
        for t in range(2):
            c = 2 * b + t
            send_bufs[t][b] = src_refs[t][b].astype(jnp.bfloat16)
            rdma = pltpu.make_async_remote_copy(
                src_ref=send_bufs[t].at[b, :, pl.ds(hl, HL)],
                dst_ref=recv_bufs[t].at[b, :, pl.ds(hl, HL)],
                send_sem=xsend_sems.at[c], recv_sem=xrecv_sems.at[c],
                device_id=partner, device_id_type=pl.DeviceIdType.MESH,
            )
            rdma.start()
            x_rdmas.append(rdma)

    y_rdmas = []
    for b in range(B):
        for t in range(2):
            c = 2 * b + t
            x_rdmas[c].wait_recv()
            fwd = pltpu.make_async_remote_copy(
                src_ref=recv_bufs[t].at[b, :, pl.ds(hl, HL)],
                dst_ref=recv_bufs[t].at[b, :, pl.ds(hl, HL)],
                send_sem=ysend_sems.at[c], recv_sem=yrecv_sems.at[c],
                device_id=peer, device_id_type=pl.DeviceIdType.MESH,
            )
            fwd.start()
            y_rdmas.append(fwd)
        for t in range(2):
            y_rdmas[2 * b + t].wait_recv()

        for h in range(H):
            lo = 64 * h
            q = (q_ref[b, :, lo:lo + 64] * SCALE).astype(jnp.bfloat16)
            k0 = ksend[b, :, lo:lo + 64]
            v0 = vsend[b, :, lo:lo + 64]
            k1 = krecv[b, :, lo:lo + 64]
            v1 = vrecv[b, :, lo:lo + 64]
            s0 = lax.dot_general(q, k0, (((1,), (1,)), ((), ())),
                                 preferred_element_type=jnp.float32)
            s1 = lax.dot_general(q, k1, (((1,), (1,)), ((), ())),
                                 preferred_element_type=jnp.float32)
            e0 = jnp.exp(s0)
            e1 = jnp.exp(s1)
            l = jnp.sum(e0, 1, keepdims=True) + jnp.sum(e1, 1, keepdims=True)
            o = (lax.dot_general(e0.astype(jnp.bfloat16), v0,
                                 (((1,), (0,)), ((), ())),
                                 preferred_element_type=jnp.float32)
                 + lax.dot_general(e1.astype(jnp.bfloat16), v1,
                                   (((1,), (0,)), ((), ())),
                                   preferred_element_type=jnp.float32))
            o_ref[b, :, lo:lo + 64] = o / l

    for r in x_rdmas:
        r.wait_send()
    for r in y_rdmas:
        r.wait_send()


def kernel(Q, K, V):
    Qr = Q.reshape(B, SQ, HD)
    Kr = K.reshape(B, SQ, HD)
    Vr = V.reshape(B, SQ, HD)

    vm = pl.BlockSpec(memory_space=pltpu.VMEM)
    out = pl.pallas_call(
        _fused_body,
        out_shape=jax.ShapeDtypeStruct((B, SQ, HD), jnp.float32),
        in_specs=[vm, vm, vm],
        out_specs=vm,
        scratch_shapes=[
            pltpu.VMEM((B, SQ, HD), jnp.bfloat16),
            pltpu.VMEM((B, SQ, HD), jnp.bfloat16),
            pltpu.VMEM((B, SQ, HD), jnp.bfloat16),
            pltpu.VMEM((B, SQ, HD), jnp.bfloat16),
            pltpu.SemaphoreType.DMA((NC,)),
            pltpu.SemaphoreType.DMA((NC,)),
            pltpu.SemaphoreType.DMA((NC,)),
            pltpu.SemaphoreType.DMA((NC,)),
        ],
        compiler_params=pltpu.CompilerParams(collective_id=0),
    )(Qr, Kr, Vr)
    return out.reshape(B, SQ, H, D)
